# baseline (speedup 1.0000x reference)
import functools

import jax
import jax.numpy as jnp
from jax import lax
from jax.experimental import pallas as pl
from jax.experimental.pallas import tpu as pltpu
from jax.experimental.pallas import tpu_sc as plsc

N = 10000
R = 10240
E = 320000
CHUNK = 128
DP = 128
NCORE = 2
NSUB = 16
NW = NCORE * NSUB
CPW = 80
EPAD = CPW * NW * CHUNK
ROWS_PW = R // NSUB
BR = 1024


def _spmm_body(p_hbm, src_hbm, dst_hbm, zero_hbm, out_hbm, src_v, dst_v, buf,
               acc):
    c = lax.axis_index("c")
    s = lax.axis_index("s")
    pltpu.sync_copy(zero_hbm.at[pl.ds(s * ROWS_PW, ROWS_PW)],
                    acc.at[pl.ds(s * ROWS_PW, ROWS_PW)])
    pltpu.sync_copy(src_hbm.at[c, s], src_v)
    pltpu.sync_copy(dst_hbm.at[c, s], dst_v)
    plsc.subcore_barrier()

    def body(j, carry):
        pltpu.sync_copy(p_hbm.at[src_v.at[j]], buf)
        pltpu.sync_copy(buf, acc.at[dst_v.at[j]], add=True)
        return carry

    lax.fori_loop(0, CPW, body, 0)
    plsc.subcore_barrier()
    pltpu.sync_copy(acc.at[pl.ds(s * ROWS_PW, ROWS_PW)],
                    out_hbm.at[c, pl.ds(s * ROWS_PW, ROWS_PW)])


@functools.lru_cache(maxsize=None)
def _make_spmm():
    mesh = plsc.VectorSubcoreMesh(core_axis_name="c", subcore_axis_name="s")
    return functools.partial(
        pl.kernel,
        mesh=mesh,
        out_type=jax.ShapeDtypeStruct((NCORE, R, DP), jnp.float32),
        scratch_types=[
            pltpu.VMEM((CPW, CHUNK), jnp.int32),
            pltpu.VMEM((CPW, CHUNK), jnp.int32),
            pltpu.VMEM((CHUNK, DP), jnp.float32),
            pltpu.VMEM_SHARED((R, DP), jnp.float32),
        ],
    )(_spmm_body)


def _deg_body(ones_hbm, dst_hbm, zero_hbm, out_hbm, dst_v, buf, acc):
    c = lax.axis_index("c")
    s = lax.axis_index("s")
    pltpu.sync_copy(zero_hbm.at[pl.ds(s * ROWS_PW, ROWS_PW)],
                    acc.at[pl.ds(s * ROWS_PW, ROWS_PW)])
    pltpu.sync_copy(dst_hbm.at[c, s], dst_v)
    pltpu.sync_copy(ones_hbm, buf)
    plsc.subcore_barrier()

    def body(j, carry):
        pltpu.sync_copy(buf, acc.at[dst_v.at[j]], add=True)
        return carry

    lax.fori_loop(0, CPW, body, 0)
    plsc.subcore_barrier()
    pltpu.sync_copy(acc.at[pl.ds(s * ROWS_PW, ROWS_PW)],
                    out_hbm.at[c, pl.ds(s * ROWS_PW, ROWS_PW)])


@functools.lru_cache(maxsize=None)
def _make_deg():
    mesh = plsc.VectorSubcoreMesh(core_axis_name="c", subcore_axis_name="s")
    return functools.partial(
        pl.kernel,
        mesh=mesh,
        out_type=jax.ShapeDtypeStruct((NCORE, R, DP), jnp.float32),
        scratch_types=[
            pltpu.VMEM((CPW, CHUNK), jnp.int32),
            pltpu.VMEM((CHUNK, DP), jnp.float32),
            pltpu.VMEM_SHARED((R, DP), jnp.float32),
        ],
    )(_deg_body)


def _dinv_of(deg0_ref, deg1_ref):
    return lax.rsqrt(deg0_ref[:, 0:1] + deg1_ref[:, 0:1] + 1.0)


def _first_body(x_ref, w_ref, deg0_ref, deg1_ref, out_ref):
    dinv = _dinv_of(deg0_ref, deg1_ref)
    out_ref[...] = jnp.dot(x_ref[...], w_ref[...],
                           preferred_element_type=jnp.float32) * dinv


def _mid_body(s0_ref, s1_ref, p_ref, deg0_ref, deg1_ref, b_ref, w_ref, out_ref,
              *, act):
    dinv = _dinv_of(deg0_ref, deg1_ref)
    h = (s0_ref[...] + s1_ref[...] + p_ref[...]) * dinv + b_ref[...]
    if act == "relu":
        h = jnp.maximum(h, 0.0)
    out_ref[...] = jnp.dot(h, w_ref[...],
                           preferred_element_type=jnp.float32) * dinv


def _last_body(s0_ref, s1_ref, p_ref, deg0_ref, deg1_ref, b_ref, out_ref, *, act):
    dinv = _dinv_of(deg0_ref, deg1_ref)
    h = (s0_ref[...] + s1_ref[...] + p_ref[...]) * dinv + b_ref[...]
    if act == "relu":
        h = jnp.maximum(h, 0.0)
    else:
        h = jax.nn.sigmoid(h)
    out_ref[...] = h


def _row_spec(d):
    return pl.BlockSpec((BR, d), lambda i: (i, 0))


def _full_spec(r, c):
    return pl.BlockSpec((r, c), lambda i: (0, 0))


def _tc_first(xp, w, deg0, deg1):
    dout = w.shape[1]
    return pl.pallas_call(
        _first_body,
        grid=(R // BR,),
        in_specs=[_row_spec(xp.shape[1]), _full_spec(*w.shape),
                  _row_spec(DP), _row_spec(DP)],
        out_specs=_row_spec(dout),
        out_shape=jax.ShapeDtypeStruct((R, dout), jnp.float32),
    )(xp, w, deg0, deg1)


def _tc_mid(s, p, deg0, deg1, b, w, act):
    dprev = p.shape[1]
    dout = w.shape[1]
    return pl.pallas_call(
        functools.partial(_mid_body, act=act),
        grid=(R // BR,),
        in_specs=[_row_spec(dprev), _row_spec(dprev), _row_spec(dprev),
                  _row_spec(DP), _row_spec(DP),
                  _full_spec(1, dprev), _full_spec(*w.shape)],
        out_specs=_row_spec(dout),
        out_shape=jax.ShapeDtypeStruct((R, dout), jnp.float32),
    )(s[0], s[1], p, deg0, deg1, b, w)


def _tc_last(s, p, deg0, deg1, b, act):
    dprev = p.shape[1]
    return pl.pallas_call(
        functools.partial(_last_body, act=act),
        grid=(R // BR,),
        in_specs=[_row_spec(dprev), _row_spec(dprev), _row_spec(dprev),
                  _row_spec(DP), _row_spec(DP), _full_spec(1, dprev)],
        out_specs=_row_spec(dprev),
        out_shape=jax.ShapeDtypeStruct((R, dprev), jnp.float32),
    )(s[0], s[1], p, deg0, deg1, b)


def _pad_w(w):
    return jnp.pad(w, ((0, DP - w.shape[0]), (0, DP - w.shape[1])))


def _pad_b(b):
    return jnp.pad(b, (0, DP - b.shape[0])).reshape(1, DP)


def _chunk_idx(v):
    pad = jnp.full((EPAD - E,), N, v.dtype)
    return jnp.concatenate([v, pad]).reshape(NCORE, NSUB, CPW, CHUNK)


def kernel(x, edge_index, W1e, b1e, W2e, b2e, Wmue, bmue, Wlse, blse, W4e, b4e,
           W1n, b1n, Wmun, bmun, Wlsn, blsn, W5n, b5n):
    del Wlse, blse, Wlsn, blsn
    xp = jnp.pad(x, ((0, R - N), (0, 0)))
    srcs = _chunk_idx(edge_index[0].astype(jnp.int32))
    dsts = _chunk_idx(edge_index[1].astype(jnp.int32))
    zero = jnp.zeros((R, DP), jnp.float32)

    degS = _make_deg()(jnp.ones((CHUNK, DP), jnp.float32), dsts, zero)
    deg0, deg1 = degS[0], degS[1]

    def spmm(p):
        return _make_spmm()(p, srcs, dsts, zero)

    p = _tc_first(xp, _pad_w(W1e), deg0, deg1)
    s = spmm(p)
    p = _tc_mid(s, p, deg0, deg1, _pad_b(b1e), _pad_w(W2e), "relu")
    s = spmm(p)
    p = _tc_mid(s, p, deg0, deg1, _pad_b(b2e), _pad_w(Wmue), "relu")
    s = spmm(p)
    p = _tc_mid(s, p, deg0, deg1, _pad_b(bmue), _pad_w(W4e), "id")
    s = spmm(p)
    edges = _tc_last(s, p, deg0, deg1, _pad_b(b4e), "sigmoid")[:N, :16]

    p = _tc_first(xp, W1n, deg0, deg1)
    s = spmm(p)
    p = _tc_mid(s, p, deg0, deg1, _pad_b(b1n), Wmun, "relu")
    s = spmm(p)
    p = _tc_mid(s, p, deg0, deg1, _pad_b(bmun), W5n, "id")
    s = spmm(p)
    nodes = _tc_last(s, p, deg0, deg1, _pad_b(b5n), "relu")[:N]

    return (edges, nodes)


# speedup vs baseline: 2.9187x; 2.9187x over previous
import functools

import jax
import jax.numpy as jnp
from jax import lax
from jax.experimental import pallas as pl
from jax.experimental.pallas import tpu as pltpu
from jax.experimental.pallas import tpu_sc as plsc

N = 10000
R = 10240
E = 320000
CHUNK = 128
DP = 128
NCORE = 2
NSUB = 16
NW = NCORE * NSUB
CPW = 80
EPAD = CPW * NW * CHUNK
ROWS_PW = R // NSUB
BR = 1024


def _spmm_body(p_hbm, src_hbm, dst_hbm, zero_hbm, out_hbm, src_v, dst_v, buf,
               acc):
    c = lax.axis_index("c")
    s = lax.axis_index("s")
    pltpu.sync_copy(zero_hbm.at[pl.ds(s * ROWS_PW, ROWS_PW)],
                    acc.at[pl.ds(s * ROWS_PW, ROWS_PW)])
    pltpu.sync_copy(src_hbm.at[c, s], src_v)
    pltpu.sync_copy(dst_hbm.at[c, s], dst_v)
    plsc.subcore_barrier()

    def body(j, carry):
        pltpu.sync_copy(p_hbm.at[src_v.at[j]], buf)
        pltpu.sync_copy(buf, acc.at[dst_v.at[j]], add=True)
        return carry

    lax.fori_loop(0, CPW, body, 0)
    plsc.subcore_barrier()
    pltpu.sync_copy(acc.at[pl.ds(s * ROWS_PW, ROWS_PW)],
                    out_hbm.at[c, pl.ds(s * ROWS_PW, ROWS_PW)])


@functools.lru_cache(maxsize=None)
def _make_spmm():
    mesh = plsc.VectorSubcoreMesh(core_axis_name="c", subcore_axis_name="s")
    return functools.partial(
        pl.kernel,
        mesh=mesh,
        out_type=jax.ShapeDtypeStruct((NCORE, R, DP), jnp.float32),
        scratch_types=[
            pltpu.VMEM((CPW, CHUNK), jnp.int32),
            pltpu.VMEM((CPW, CHUNK), jnp.int32),
            pltpu.VMEM((CHUNK, DP), jnp.float32),
            pltpu.VMEM_SHARED((R, DP), jnp.float32),
        ],
    )(_spmm_body)


def _deg_body(ones_hbm, dst_hbm, zero_hbm, out_hbm, dst_v, buf, acc):
    c = lax.axis_index("c")
    s = lax.axis_index("s")
    pltpu.sync_copy(zero_hbm.at[pl.ds(s * ROWS_PW, ROWS_PW)],
                    acc.at[pl.ds(s * ROWS_PW, ROWS_PW)])
    pltpu.sync_copy(dst_hbm.at[c, s], dst_v)
    pltpu.sync_copy(ones_hbm, buf)
    plsc.subcore_barrier()

    def body(j, carry):
        pltpu.sync_copy(buf, acc.at[dst_v.at[j]], add=True)
        return carry

    lax.fori_loop(0, CPW, body, 0)
    plsc.subcore_barrier()
    pltpu.sync_copy(acc.at[pl.ds(s * ROWS_PW, ROWS_PW)],
                    out_hbm.at[c, pl.ds(s * ROWS_PW, ROWS_PW)])


@functools.lru_cache(maxsize=None)
def _make_deg():
    mesh = plsc.VectorSubcoreMesh(core_axis_name="c", subcore_axis_name="s")
    return functools.partial(
        pl.kernel,
        mesh=mesh,
        out_type=jax.ShapeDtypeStruct((NCORE, R, DP), jnp.float32),
        scratch_types=[
            pltpu.VMEM((CPW, CHUNK), jnp.int32),
            pltpu.VMEM((CHUNK, DP), jnp.float32),
            pltpu.VMEM_SHARED((R, DP), jnp.float32),
        ],
    )(_deg_body)


def _dinv_of(deg0_ref, deg1_ref):
    return lax.rsqrt(deg0_ref[:, 0:1] + deg1_ref[:, 0:1] + 1.0)


def _first_body(x_ref, w_ref, deg0_ref, deg1_ref, out_ref):
    dinv = _dinv_of(deg0_ref, deg1_ref)
    out_ref[...] = jnp.dot(x_ref[...], w_ref[...],
                           preferred_element_type=jnp.float32) * dinv


def _mid_body(s0_ref, s1_ref, p_ref, deg0_ref, deg1_ref, b_ref, w_ref, out_ref,
              *, act):
    dinv = _dinv_of(deg0_ref, deg1_ref)
    h = (s0_ref[...] + s1_ref[...] + p_ref[...]) * dinv + b_ref[...]
    if act == "relu":
        h = jnp.maximum(h, 0.0)
    out_ref[...] = jnp.dot(h, w_ref[...],
                           preferred_element_type=jnp.float32) * dinv


def _last_body(s0_ref, s1_ref, p_ref, deg0_ref, deg1_ref, b_ref, out_ref, *, act):
    dinv = _dinv_of(deg0_ref, deg1_ref)
    h = (s0_ref[...] + s1_ref[...] + p_ref[...]) * dinv + b_ref[...]
    if act == "relu":
        h = jnp.maximum(h, 0.0)
    else:
        h = jax.nn.sigmoid(h)
    out_ref[...] = h


def _row_spec(d):
    return pl.BlockSpec((BR, d), lambda i: (i, 0))


def _full_spec(r, c):
    return pl.BlockSpec((r, c), lambda i: (0, 0))


def _tc_first(xp, w, deg0, deg1):
    dout = w.shape[1]
    return pl.pallas_call(
        _first_body,
        grid=(R // BR,),
        in_specs=[_row_spec(xp.shape[1]), _full_spec(*w.shape),
                  _row_spec(DP), _row_spec(DP)],
        out_specs=_row_spec(dout),
        out_shape=jax.ShapeDtypeStruct((R, dout), jnp.float32),
    )(xp, w, deg0, deg1)


def _tc_mid(s, p, deg0, deg1, b, w, act):
    dprev = p.shape[1]
    dout = w.shape[1]
    return pl.pallas_call(
        functools.partial(_mid_body, act=act),
        grid=(R // BR,),
        in_specs=[_row_spec(dprev), _row_spec(dprev), _row_spec(dprev),
                  _row_spec(DP), _row_spec(DP),
                  _full_spec(1, dprev), _full_spec(*w.shape)],
        out_specs=_row_spec(dout),
        out_shape=jax.ShapeDtypeStruct((R, dout), jnp.float32),
    )(s[0], s[1], p, deg0, deg1, b, w)


def _tc_last(s, p, deg0, deg1, b, act):
    dprev = p.shape[1]
    return pl.pallas_call(
        functools.partial(_last_body, act=act),
        grid=(R // BR,),
        in_specs=[_row_spec(dprev), _row_spec(dprev), _row_spec(dprev),
                  _row_spec(DP), _row_spec(DP), _full_spec(1, dprev)],
        out_specs=_row_spec(dprev),
        out_shape=jax.ShapeDtypeStruct((R, dprev), jnp.float32),
    )(s[0], s[1], p, deg0, deg1, b)


def _pad_w(w):
    return jnp.pad(w, ((0, DP - w.shape[0]), (0, DP - w.shape[1])))


def _pad_b(b):
    return jnp.pad(b, (0, DP - b.shape[0])).reshape(1, DP)


def _chunk_idx(v):
    pad = N + (jnp.arange(EPAD - E, dtype=v.dtype) % (R - N))
    return jnp.concatenate([v, pad]).reshape(NCORE, NSUB, CPW, CHUNK)


def kernel(x, edge_index, W1e, b1e, W2e, b2e, Wmue, bmue, Wlse, blse, W4e, b4e,
           W1n, b1n, Wmun, bmun, Wlsn, blsn, W5n, b5n):
    del Wlse, blse, Wlsn, blsn
    xp = jnp.pad(x, ((0, R - N), (0, 0)))
    srcs = _chunk_idx(edge_index[0].astype(jnp.int32))
    dsts = _chunk_idx(edge_index[1].astype(jnp.int32))
    zero = jnp.zeros((R, DP), jnp.float32)

    degS = _make_deg()(jnp.ones((CHUNK, DP), jnp.float32), dsts, zero)
    deg0, deg1 = degS[0], degS[1]

    def spmm(p):
        return _make_spmm()(p, srcs, dsts, zero)

    p = _tc_first(xp, _pad_w(W1e), deg0, deg1)
    s = spmm(p)
    p = _tc_mid(s, p, deg0, deg1, _pad_b(b1e), _pad_w(W2e), "relu")
    s = spmm(p)
    p = _tc_mid(s, p, deg0, deg1, _pad_b(b2e), _pad_w(Wmue), "relu")
    s = spmm(p)
    p = _tc_mid(s, p, deg0, deg1, _pad_b(bmue), _pad_w(W4e), "id")
    s = spmm(p)
    edges = _tc_last(s, p, deg0, deg1, _pad_b(b4e), "sigmoid")[:N, :16]

    p = _tc_first(xp, W1n, deg0, deg1)
    s = spmm(p)
    p = _tc_mid(s, p, deg0, deg1, _pad_b(b1n), Wmun, "relu")
    s = spmm(p)
    p = _tc_mid(s, p, deg0, deg1, _pad_b(bmun), W5n, "id")
    s = spmm(p)
    nodes = _tc_last(s, p, deg0, deg1, _pad_b(b5n), "relu")[:N]

    return (edges, nodes)


# speedup vs baseline: 4.3447x; 1.4886x over previous
import functools

import jax
import jax.numpy as jnp
from jax import lax
from jax.experimental import pallas as pl
from jax.experimental.pallas import tpu as pltpu
from jax.experimental.pallas import tpu_sc as plsc

N = 10000
R = 10240
E = 320000
CHUNK = 128
DP = 128
NCORE = 2
NSUB = 16
NW = NCORE * NSUB
BLKC = 10
NBLK = 8
CPW = BLKC * NBLK
EPAD = CPW * NW * CHUNK
ROWS_PW = R // NSUB
BR = 1024


def _spmm_body(p_hbm, idx_hbm, zero_hbm, out_hbm, idx_v, bufs, acc,
               i0, i1, g0, g1):
    isems = (i0, i1)
    gsems = (g0, g1)
    c = lax.axis_index("c")
    s = lax.axis_index("s")
    pltpu.sync_copy(zero_hbm.at[pl.ds(s * ROWS_PW, ROWS_PW)],
                    acc.at[pl.ds(s * ROWS_PW, ROWS_PW)])
    plsc.subcore_barrier()

    pltpu.async_copy(idx_hbm.at[c, s, 0], idx_v.at[0], isems[0])
    pltpu.async_copy(idx_hbm.at[c, s, 1], idx_v.at[1], isems[1])
    pltpu.make_async_copy(idx_hbm.at[c, s, 0], idx_v.at[0], isems[0]).wait()
    pltpu.async_copy(p_hbm.at[idx_v.at[0, 0, 0]], bufs.at[0], gsems[0])
    pltpu.async_copy(p_hbm.at[idx_v.at[0, 1, 0]], bufs.at[1], gsems[1])

    def block(blk, par, wait_next_isem, load_blk2, gather_limit):
        par2 = (par + 1) % 2
        for q in range(BLKC):
            b = q % 2
            pltpu.make_async_copy(p_hbm.at[idx_v.at[par, q, 0]], bufs.at[b],
                                  gsems[b]).wait()
            pltpu.sync_copy(bufs.at[b], acc.at[idx_v.at[par, q, 1]], add=True)
            if q == BLKC - 2 and wait_next_isem:
                pltpu.make_async_copy(idx_hbm.at[c, s, blk], idx_v.at[par2],
                                      isems[par2]).wait()
            if q + 2 < gather_limit:
                qq, nb = q + 2, par
                if qq >= BLKC:
                    qq, nb = qq - BLKC, par2
                pltpu.async_copy(p_hbm.at[idx_v.at[nb, qq, 0]], bufs.at[b],
                                 gsems[b])
            if q == BLKC - 1 and load_blk2:
                pltpu.async_copy(idx_hbm.at[c, s, blk + 2], idx_v.at[par],
                                 isems[par])

    def round_(k, carry):
        blk = 2 * k
        block(blk, 0, True, True, BLKC + 2)
        block(blk + 1, 1, True, True, BLKC + 2)
        return carry

    lax.fori_loop(0, NBLK // 2 - 1, round_, 0)
    block(NBLK - 2, 0, True, False, BLKC + 2)
    block(NBLK - 1, 1, False, False, BLKC)

    plsc.subcore_barrier()
    pltpu.sync_copy(acc.at[pl.ds(s * ROWS_PW, ROWS_PW)],
                    out_hbm.at[c, pl.ds(s * ROWS_PW, ROWS_PW)])


@functools.lru_cache(maxsize=None)
def _make_spmm():
    mesh = plsc.VectorSubcoreMesh(core_axis_name="c", subcore_axis_name="s")
    return functools.partial(
        pl.kernel,
        mesh=mesh,
        out_type=jax.ShapeDtypeStruct((NCORE, R, DP), jnp.float32),
        scratch_types=[
            pltpu.VMEM((2, BLKC, 2, CHUNK), jnp.int32),
            pltpu.VMEM((2, CHUNK, DP), jnp.float32),
            pltpu.VMEM_SHARED((R, DP), jnp.float32),
            pltpu.SemaphoreType.DMA,
            pltpu.SemaphoreType.DMA,
            pltpu.SemaphoreType.DMA,
            pltpu.SemaphoreType.DMA,
        ],
    )(_spmm_body)


def _deg_body(ones_hbm, dst_hbm, zero_hbm, out_hbm, dst_v, buf, acc):
    c = lax.axis_index("c")
    s = lax.axis_index("s")
    pltpu.sync_copy(zero_hbm.at[pl.ds(s * ROWS_PW, ROWS_PW)],
                    acc.at[pl.ds(s * ROWS_PW, ROWS_PW)])
    pltpu.sync_copy(dst_hbm.at[c, s], dst_v)
    pltpu.sync_copy(ones_hbm, buf)
    plsc.subcore_barrier()

    def body(j, carry):
        pltpu.sync_copy(buf, acc.at[dst_v.at[j]], add=True)
        return carry

    lax.fori_loop(0, CPW, body, 0)
    plsc.subcore_barrier()
    pltpu.sync_copy(acc.at[pl.ds(s * ROWS_PW, ROWS_PW)],
                    out_hbm.at[c, pl.ds(s * ROWS_PW, ROWS_PW)])


@functools.lru_cache(maxsize=None)
def _make_deg():
    mesh = plsc.VectorSubcoreMesh(core_axis_name="c", subcore_axis_name="s")
    return functools.partial(
        pl.kernel,
        mesh=mesh,
        out_type=jax.ShapeDtypeStruct((NCORE, R, DP), jnp.float32),
        scratch_types=[
            pltpu.VMEM((CPW, CHUNK), jnp.int32),
            pltpu.VMEM((CHUNK, DP), jnp.float32),
            pltpu.VMEM_SHARED((R, DP), jnp.float32),
        ],
    )(_deg_body)


def _dinv_of(deg0_ref, deg1_ref):
    return lax.rsqrt(deg0_ref[:, 0:1] + deg1_ref[:, 0:1] + 1.0)


def _first_body(x_ref, w_ref, deg0_ref, deg1_ref, out_ref):
    dinv = _dinv_of(deg0_ref, deg1_ref)
    out_ref[...] = jnp.dot(x_ref[...], w_ref[...],
                           preferred_element_type=jnp.float32) * dinv


def _mid_body(s0_ref, s1_ref, p_ref, deg0_ref, deg1_ref, b_ref, w_ref, out_ref,
              *, act):
    dinv = _dinv_of(deg0_ref, deg1_ref)
    h = (s0_ref[...] + s1_ref[...] + p_ref[...]) * dinv + b_ref[...]
    if act == "relu":
        h = jnp.maximum(h, 0.0)
    out_ref[...] = jnp.dot(h, w_ref[...],
                           preferred_element_type=jnp.float32) * dinv


def _last_body(s0_ref, s1_ref, p_ref, deg0_ref, deg1_ref, b_ref, out_ref, *, act):
    dinv = _dinv_of(deg0_ref, deg1_ref)
    h = (s0_ref[...] + s1_ref[...] + p_ref[...]) * dinv + b_ref[...]
    if act == "relu":
        h = jnp.maximum(h, 0.0)
    else:
        h = jax.nn.sigmoid(h)
    out_ref[...] = h


def _row_spec(d):
    return pl.BlockSpec((BR, d), lambda i: (i, 0))


def _full_spec(r, c):
    return pl.BlockSpec((r, c), lambda i: (0, 0))


def _tc_first(xp, w, deg0, deg1):
    dout = w.shape[1]
    return pl.pallas_call(
        _first_body,
        grid=(R // BR,),
        in_specs=[_row_spec(xp.shape[1]), _full_spec(*w.shape),
                  _row_spec(DP), _row_spec(DP)],
        out_specs=_row_spec(dout),
        out_shape=jax.ShapeDtypeStruct((R, dout), jnp.float32),
    )(xp, w, deg0, deg1)


def _tc_mid(s, p, deg0, deg1, b, w, act):
    dprev = p.shape[1]
    dout = w.shape[1]
    return pl.pallas_call(
        functools.partial(_mid_body, act=act),
        grid=(R // BR,),
        in_specs=[_row_spec(dprev), _row_spec(dprev), _row_spec(dprev),
                  _row_spec(DP), _row_spec(DP),
                  _full_spec(1, dprev), _full_spec(*w.shape)],
        out_specs=_row_spec(dout),
        out_shape=jax.ShapeDtypeStruct((R, dout), jnp.float32),
    )(s[0], s[1], p, deg0, deg1, b, w)


def _tc_last(s, p, deg0, deg1, b, act):
    dprev = p.shape[1]
    return pl.pallas_call(
        functools.partial(_last_body, act=act),
        grid=(R // BR,),
        in_specs=[_row_spec(dprev), _row_spec(dprev), _row_spec(dprev),
                  _row_spec(DP), _row_spec(DP), _full_spec(1, dprev)],
        out_specs=_row_spec(dprev),
        out_shape=jax.ShapeDtypeStruct((R, dprev), jnp.float32),
    )(s[0], s[1], p, deg0, deg1, b)


def _pad_w(w):
    return jnp.pad(w, ((0, DP - w.shape[0]), (0, DP - w.shape[1])))


def _pad_b(b):
    return jnp.pad(b, (0, DP - b.shape[0])).reshape(1, DP)


def _chunk_idx(v):
    pad = N + (jnp.arange(EPAD - E, dtype=v.dtype) % (R - N))
    return jnp.concatenate([v, pad]).reshape(NCORE, NSUB, CPW, CHUNK)


def kernel(x, edge_index, W1e, b1e, W2e, b2e, Wmue, bmue, Wlse, blse, W4e, b4e,
           W1n, b1n, Wmun, bmun, Wlsn, blsn, W5n, b5n):
    del Wlse, blse, Wlsn, blsn
    xp = jnp.pad(x, ((0, R - N), (0, 0)))
    srcs = _chunk_idx(edge_index[0].astype(jnp.int32))
    dsts = _chunk_idx(edge_index[1].astype(jnp.int32))
    idx = jnp.stack([srcs.reshape(NCORE, NSUB, NBLK, BLKC, CHUNK),
                     dsts.reshape(NCORE, NSUB, NBLK, BLKC, CHUNK)], axis=4)
    zero = jnp.zeros((R, DP), jnp.float32)

    degS = _make_deg()(jnp.ones((CHUNK, DP), jnp.float32), dsts, zero)
    deg0, deg1 = degS[0], degS[1]

    def spmm(p):
        return _make_spmm()(p, idx, zero)

    p = _tc_first(xp, _pad_w(W1e), deg0, deg1)
    s = spmm(p)
    p = _tc_mid(s, p, deg0, deg1, _pad_b(b1e), _pad_w(W2e), "relu")
    s = spmm(p)
    p = _tc_mid(s, p, deg0, deg1, _pad_b(b2e), _pad_w(Wmue), "relu")
    s = spmm(p)
    p = _tc_mid(s, p, deg0, deg1, _pad_b(bmue), _pad_w(W4e), "id")
    s = spmm(p)
    edges = _tc_last(s, p, deg0, deg1, _pad_b(b4e), "sigmoid")[:N, :16]

    p = _tc_first(xp, W1n, deg0, deg1)
    s = spmm(p)
    p = _tc_mid(s, p, deg0, deg1, _pad_b(b1n), Wmun, "relu")
    s = spmm(p)
    p = _tc_mid(s, p, deg0, deg1, _pad_b(bmun), W5n, "id")
    s = spmm(p)
    nodes = _tc_last(s, p, deg0, deg1, _pad_b(b5n), "relu")[:N]

    return (edges, nodes)
